# baseline (speedup 1.0000x reference)
import functools

import jax
import jax.numpy as jnp
from jax import lax
from jax.experimental import pallas as pl
from jax.experimental.pallas import tpu as pltpu
from jax.experimental.pallas import tpu_sc as plsc

D_MODEL = 1024
SEQ_LEN = 4096
BATCH = 4

NUM_WORKERS = 32
SC_ROWS = 2048
SC_ROWS_PER_WORKER = SC_ROWS // NUM_WORKERS

TC_BS = 512
TC_OFF_BLK = SC_ROWS // TC_BS
TC_NBLK = (SEQ_LEN - SC_ROWS) // TC_BS


def _sc_stage(pos_table):
    mesh = plsc.VectorSubcoreMesh(core_axis_name="c", subcore_axis_name="s")

    @functools.partial(
        pl.kernel,
        out_type=jax.ShapeDtypeStruct((BATCH, SEQ_LEN, D_MODEL), jnp.float32),
        mesh=mesh,
        scratch_types=[
            pltpu.VMEM((SC_ROWS_PER_WORKER, D_MODEL), jnp.float32),
            pltpu.SemaphoreType.DMA,
        ],
    )
    def body(pos_hbm, out_hbm, buf, sem):
        wid = lax.axis_index("s") * mesh.num_cores + lax.axis_index("c")
        base = wid * SC_ROWS_PER_WORKER
        pltpu.sync_copy(pos_hbm.at[pl.ds(base, SC_ROWS_PER_WORKER)], buf)
        copies = [
            pltpu.async_copy(
                buf, out_hbm.at[b, pl.ds(base, SC_ROWS_PER_WORKER)], sem
            )
            for b in range(BATCH)
        ]
        for c in copies:
            c.wait()

    return body(pos_table)


def _tc_stage(pos_table, partial_out):
    def body(pos_ref, aliased_ref, out_ref):
        del aliased_ref
        out_ref[0] = pos_ref[...]

    return pl.pallas_call(
        body,
        grid=(TC_NBLK, BATCH),
        in_specs=[
            pl.BlockSpec((TC_BS, D_MODEL), lambda j, b: (TC_OFF_BLK + j, 0)),
            pl.BlockSpec(memory_space=pltpu.HBM),
        ],
        out_specs=pl.BlockSpec(
            (1, TC_BS, D_MODEL), lambda j, b: (b, TC_OFF_BLK + j, 0)
        ),
        out_shape=jax.ShapeDtypeStruct((BATCH, SEQ_LEN, D_MODEL), jnp.float32),
        input_output_aliases={1: 0},
    )(pos_table, partial_out)


def kernel(x, pos_table):
    del x
    return _tc_stage(pos_table, _sc_stage(pos_table))


# speedup vs baseline: 1.1373x; 1.1373x over previous
import functools

import jax
import jax.numpy as jnp
from jax import lax
from jax.experimental import pallas as pl
from jax.experimental.pallas import tpu as pltpu
from jax.experimental.pallas import tpu_sc as plsc

D_MODEL = 1024
SEQ_LEN = 4096
BATCH = 4
NUM_WORKERS = 32
ROWS_PER_WORKER = SEQ_LEN // NUM_WORKERS
CHUNK = 64


def _sc_broadcast(pos_table):
    mesh = plsc.VectorSubcoreMesh(core_axis_name="c", subcore_axis_name="s")

    @functools.partial(
        pl.kernel,
        out_type=jax.ShapeDtypeStruct((BATCH, SEQ_LEN, D_MODEL), jnp.float32),
        mesh=mesh,
        scratch_types=[
            pltpu.VMEM((CHUNK, D_MODEL), jnp.float32),
            pltpu.SemaphoreType.DMA,
        ],
    )
    def body(pos_hbm, out_hbm, buf, sem):
        wid = lax.axis_index("s") * mesh.num_cores + lax.axis_index("c")
        base = wid * ROWS_PER_WORKER
        for c in range(ROWS_PER_WORKER // CHUNK):
            r0 = base + c * CHUNK
            pltpu.sync_copy(pos_hbm.at[pl.ds(r0, CHUNK)], buf)
            copies = [
                pltpu.async_copy(buf, out_hbm.at[b, pl.ds(r0, CHUNK)], sem)
                for b in range(BATCH)
            ]
            for cc in copies:
                cc.wait()

    return body(pos_table)


def kernel(x, pos_table):
    del x
    return _sc_broadcast(pos_table)
